# baseline (speedup 1.0000x reference)
import functools

import jax
import jax.numpy as jnp
from jax.experimental import pallas as pl
from jax.experimental.pallas import tpu as pltpu

PATCH = 32


def _round_up(x, m):
    return ((x + m - 1) // m) * m


def _patchify_kernel(x_ref, w_ref, bias_ref, wih_ref, feat_ref, gx_ref, *,
                     C, Hf, Wf, nf):
    rows = nf * Hf
    K = x_ref.shape[-1]
    F = w_ref.shape[-1]
    lane = jax.lax.broadcasted_iota(jnp.int32, (1, K), 1)
    m0 = ((lane % (Wf * PATCH)) < PATCH).astype(jnp.bfloat16)
    m1 = jnp.bfloat16(1.0) - m0

    xbh = x_ref[...].astype(jnp.bfloat16)
    ppq = PATCH // x_ref.shape[3]
    acc0 = jnp.zeros((rows, F), jnp.float32)
    acc1 = jnp.zeros((rows, F), jnp.float32)
    for c in range(C):
        for q in range(x_ref.shape[3]):
            a = xbh[:, c, :, q, :].reshape(rows, K)
            slabs = []
            for j in range(ppq):
                s = w_ref[c, ppq * q + j]
                slabs += [s, s]
            b_eff = jnp.concatenate(slabs, axis=0)
            acc0 += jnp.dot(a * m0, b_eff,
                            preferred_element_type=jnp.float32)
            acc1 += jnp.dot(a * m1, b_eff,
                            preferred_element_type=jnp.float32)

    bias = bias_ref[...]
    r0 = acc0 + bias
    r0 = r0 * jax.nn.sigmoid(r0)
    r1 = acc1 + bias
    r1 = r1 * jax.nn.sigmoid(r1)
    feat_ref[:, :, :, 0, :] = r0.reshape(nf, Hf, F // 128, 128)
    feat_ref[:, :, :, 1, :] = r1.reshape(nf, Hf, F // 128, 128)
    pooled = ((r0 + r1).reshape(nf, Hf, F).sum(axis=1)
              * (1.0 / (Hf * Wf)))
    gx_ref[...] = jnp.dot(pooled, wih_ref[...],
                          preferred_element_type=jnp.float32)


def _patchify_gx(x, w4, bias, wih, nf):
    N, C, Hf, Q2, L = x.shape
    Wf = L * Q2 // (PATCH * PATCH)
    F = w4.shape[-1]
    G = wih.shape[1]
    grid = (N // nf,)
    return pl.pallas_call(
        functools.partial(_patchify_kernel, C=C, Hf=Hf, Wf=Wf, nf=nf),
        out_shape=(jax.ShapeDtypeStruct((N, Hf, F // 128, Wf, 128),
                                        jnp.float32),
                   jax.ShapeDtypeStruct((N, G), jnp.float32)),
        grid=grid,
        in_specs=[
            pl.BlockSpec((nf, C, Hf, Q2, L), lambda i: (i, 0, 0, 0, 0)),
            pl.BlockSpec((C, PATCH, PATCH, F), lambda i: (0, 0, 0, 0)),
            pl.BlockSpec((1, F), lambda i: (0, 0)),
            pl.BlockSpec((F, G), lambda i: (0, 0)),
        ],
        out_specs=[
            pl.BlockSpec((nf, Hf, F // 128, Wf, 128),
                         lambda i: (i, 0, 0, 0, 0)),
            pl.BlockSpec((nf, G), lambda i: (i, 0)),
        ],
        compiler_params=pltpu.CompilerParams(
            dimension_semantics=("parallel",),
            vmem_limit_bytes=48 * 1024 * 1024),
    )(x, w4, bias, wih)


def _lstm_head_kernel(gx_ref, whh_ref, wlin_ref, blin_ref, logits_ref,
                      mrow_ref, *, T, NB):
    H = whh_ref.shape[0]
    whh = whh_ref[...]

    def step(t, carry):
        h, c = carry
        g = gx_ref[pl.ds(t * NB, NB), :] + jnp.dot(
            h, whh, preferred_element_type=jnp.float32)
        i = jax.nn.sigmoid(g[:, 0 * H:1 * H])
        f = jax.nn.sigmoid(g[:, 1 * H:2 * H])
        gg = jnp.tanh(g[:, 2 * H:3 * H])
        o = jax.nn.sigmoid(g[:, 3 * H:4 * H])
        c = f * c + i * gg
        h = o * jnp.tanh(c)
        mrow_ref[pl.ds(t, 1), :] = jnp.mean(h, axis=0, keepdims=True)
        return h, c

    h0 = jnp.zeros((NB, H), jnp.float32)
    jax.lax.fori_loop(0, T, step, (h0, h0))
    logits_ref[...] = (jnp.dot(mrow_ref[...], wlin_ref[...],
                               preferred_element_type=jnp.float32)
                       + blin_ref[...])


def _lstm_head(gx, whh, wlin_p, blin_p, T, NB):
    G = gx.shape[1]
    H = whh.shape[0]
    NCp = wlin_p.shape[1]
    return pl.pallas_call(
        functools.partial(_lstm_head_kernel, T=T, NB=NB),
        out_shape=jax.ShapeDtypeStruct((T, NCp), jnp.float32),
        grid=(1,),
        in_specs=[
            pl.BlockSpec((T * NB, G), lambda i: (0, 0)),
            pl.BlockSpec((H, G), lambda i: (0, 0)),
            pl.BlockSpec((H, NCp), lambda i: (0, 0)),
            pl.BlockSpec((1, NCp), lambda i: (0, 0)),
        ],
        out_specs=pl.BlockSpec((T, NCp), lambda i: (0, 0)),
        scratch_shapes=[pltpu.VMEM((T, H), jnp.float32)],
        compiler_params=pltpu.CompilerParams(
            dimension_semantics=("arbitrary",)),
    )(gx, whh, wlin_p, blin_p)


def kernel(x, w_feat, b_feat, w_ih_t, w_hh_t, w_lin_t, b_lin):
    B, S, C, H, W = x.shape
    N = B * S
    Hf, Wf = H // PATCH, W // PATCH
    P = Hf * Wf
    F = w_feat.shape[1]

    xv = x.reshape(N, C, Hf, PATCH // 4, 4 * Wf * PATCH)
    w4 = w_feat.astype(jnp.bfloat16).reshape(C, PATCH, PATCH, F)
    bias = b_feat.astype(jnp.float32).reshape(1, F)
    wih = w_ih_t.astype(jnp.float32)

    nf = 32 if N % 32 == 0 else N
    feat5, gx = _patchify_gx(xv, w4, bias, wih, nf)
    fmap = feat5.transpose(0, 2, 4, 1, 3).reshape(N, F, Hf, Wf)

    NC = w_lin_t.shape[1]
    NCp = _round_up(NC, 128)
    wlin_p = jnp.pad(w_lin_t.astype(jnp.float32), ((0, 0), (0, NCp - NC)))
    blin_p = jnp.pad(b_lin.astype(jnp.float32), (0, NCp - NC)).reshape(1, NCp)

    logits = _lstm_head(gx, w_hh_t.astype(jnp.float32), wlin_p, blin_p,
                        T=B, NB=S)[:, :NC]
    return fmap, logits


# speedup vs baseline: 1.4105x; 1.4105x over previous
import functools

import jax
import jax.numpy as jnp
from jax.experimental import pallas as pl
from jax.experimental.pallas import tpu as pltpu

PATCH = 32


def _round_up(x, m):
    return ((x + m - 1) // m) * m


def _patchify_kernel(x_ref, w_ref, bias_ref, wih_ref, feat_ref, gx_ref, *,
                     C, Hf, Wf, nf):
    rows = nf * Hf
    K = x_ref.shape[-1]
    F = w_ref.shape[-1]
    lane = jax.lax.broadcasted_iota(jnp.int32, (1, K), 1)
    m0 = ((lane % (Wf * PATCH)) < PATCH).astype(jnp.bfloat16)
    m1 = jnp.bfloat16(1.0) - m0

    xbh = x_ref[...].astype(jnp.bfloat16)
    ppq = PATCH // x_ref.shape[3]
    acc0 = jnp.zeros((rows, F), jnp.float32)
    acc1 = jnp.zeros((rows, F), jnp.float32)
    for c in range(C):
        for q in range(x_ref.shape[3]):
            a = xbh[:, c, :, q, :].reshape(rows, K)
            slabs = []
            for j in range(ppq):
                s = w_ref[c, ppq * q + j]
                slabs += [s, s]
            b_eff = jnp.concatenate(slabs, axis=0)
            acc0 += jnp.dot(a * m0, b_eff,
                            preferred_element_type=jnp.float32)
            acc1 += jnp.dot(a * m1, b_eff,
                            preferred_element_type=jnp.float32)

    bias = bias_ref[...]
    r0 = acc0 + bias
    r0 = r0 * jax.nn.sigmoid(r0)
    r1 = acc1 + bias
    r1 = r1 * jax.nn.sigmoid(r1)
    feat_ref[:, :, :, 0, :] = r0.reshape(nf, Hf, F // 128, 128)
    feat_ref[:, :, :, 1, :] = r1.reshape(nf, Hf, F // 128, 128)
    pooled = ((r0 + r1).reshape(nf, Hf, F).sum(axis=1)
              * (1.0 / (Hf * Wf)))
    gx_ref[...] = jnp.dot(pooled, wih_ref[...],
                          preferred_element_type=jnp.float32)


def _patchify_gx(x, w4, bias, wih, nf):
    N, C, Hf, Q2, L = x.shape
    Wf = L * Q2 // (PATCH * PATCH)
    F = w4.shape[-1]
    G = wih.shape[1]
    grid = (N // nf,)
    return pl.pallas_call(
        functools.partial(_patchify_kernel, C=C, Hf=Hf, Wf=Wf, nf=nf),
        out_shape=(jax.ShapeDtypeStruct((N, Hf, F // 128, Wf, 128),
                                        jnp.float32),
                   jax.ShapeDtypeStruct((N, G), jnp.float32)),
        grid=grid,
        in_specs=[
            pl.BlockSpec((nf, C, Hf, Q2, L), lambda i: (i, 0, 0, 0, 0)),
            pl.BlockSpec((C, PATCH, PATCH, F), lambda i: (0, 0, 0, 0)),
            pl.BlockSpec((1, F), lambda i: (0, 0)),
            pl.BlockSpec((F, G), lambda i: (0, 0)),
        ],
        out_specs=[
            pl.BlockSpec((nf, Hf, F // 128, Wf, 128),
                         lambda i: (i, 0, 0, 0, 0)),
            pl.BlockSpec((nf, G), lambda i: (i, 0)),
        ],
        compiler_params=pltpu.CompilerParams(
            dimension_semantics=("parallel",),
            vmem_limit_bytes=48 * 1024 * 1024),
    )(x, w4, bias, wih)


def _lstm_head_kernel(gx_ref, whh_ref, wlin_ref, blin_ref, logits_ref,
                      mrow_ref, *, T, NB):
    H = whh_ref.shape[0]
    whh = whh_ref[...]

    def step(t, carry):
        h, c = carry
        g = gx_ref[pl.ds(t * NB, NB), :] + jnp.dot(
            h, whh, preferred_element_type=jnp.float32)
        i = jax.nn.sigmoid(g[:, 0 * H:1 * H])
        f = jax.nn.sigmoid(g[:, 1 * H:2 * H])
        gg = jnp.tanh(g[:, 2 * H:3 * H])
        o = jax.nn.sigmoid(g[:, 3 * H:4 * H])
        c = f * c + i * gg
        h = o * jnp.tanh(c)
        mrow_ref[pl.ds(t, 1), :] = jnp.mean(h, axis=0, keepdims=True)
        return h, c

    h0 = jnp.zeros((NB, H), jnp.float32)
    jax.lax.fori_loop(0, T, step, (h0, h0))
    logits_ref[...] = (jnp.dot(mrow_ref[...], wlin_ref[...],
                               preferred_element_type=jnp.float32)
                       + blin_ref[...])


def _lstm_head(gx, whh, wlin_p, blin_p, T, NB):
    G = gx.shape[1]
    H = whh.shape[0]
    NCp = wlin_p.shape[1]
    return pl.pallas_call(
        functools.partial(_lstm_head_kernel, T=T, NB=NB),
        out_shape=jax.ShapeDtypeStruct((T, NCp), jnp.float32),
        grid=(1,),
        in_specs=[
            pl.BlockSpec((T * NB, G), lambda i: (0, 0)),
            pl.BlockSpec((H, G), lambda i: (0, 0)),
            pl.BlockSpec((H, NCp), lambda i: (0, 0)),
            pl.BlockSpec((1, NCp), lambda i: (0, 0)),
        ],
        out_specs=pl.BlockSpec((T, NCp), lambda i: (0, 0)),
        scratch_shapes=[pltpu.VMEM((T, H), jnp.float32)],
        compiler_params=pltpu.CompilerParams(
            dimension_semantics=("arbitrary",)),
    )(gx, whh, wlin_p, blin_p)


def kernel(x, w_feat, b_feat, w_ih_t, w_hh_t, w_lin_t, b_lin):
    B, S, C, H, W = x.shape
    N = B * S
    Hf, Wf = H // PATCH, W // PATCH
    P = Hf * Wf
    F = w_feat.shape[1]

    xv = x.reshape(N, C, Hf, PATCH // 4, 4 * Wf * PATCH)
    w4 = w_feat.astype(jnp.bfloat16).reshape(C, PATCH, PATCH, F)
    bias = b_feat.astype(jnp.float32).reshape(1, F)
    wih = w_ih_t.astype(jnp.float32)

    nf = 64 if N % 64 == 0 else N
    feat5, gx = _patchify_gx(xv, w4, bias, wih, nf)
    fmap = feat5.transpose(0, 2, 4, 1, 3).reshape(N, F, Hf, Wf)

    NC = w_lin_t.shape[1]
    NCp = _round_up(NC, 128)
    wlin_p = jnp.pad(w_lin_t.astype(jnp.float32), ((0, 0), (0, NCp - NC)))
    blin_p = jnp.pad(b_lin.astype(jnp.float32), (0, NCp - NC)).reshape(1, NCp)

    logits = _lstm_head(gx, w_hh_t.astype(jnp.float32), wlin_p, blin_p,
                        T=B, NB=S)[:, :NC]
    return fmap, logits
